# baseline (speedup 1.0000x reference)
import functools
import jax
import jax.numpy as jnp
from jax import lax
from jax.experimental import pallas as pl
from jax.experimental.pallas import tpu as pltpu
from jax.experimental.pallas import tpu_sc as plsc

D_MODEL = 64
SCALE = 8.0
LANES = 16
CHUNK = 128
NBUF = 4


def kernel(lut, x):
    batch_shape = x.shape
    xf = x.reshape(-1).astype(jnp.int32)
    total = xf.shape[0]
    lut_pairs = lut.reshape(-1, 2 * D_MODEL)

    info = plsc.get_sparse_core_info()
    num_workers = info.num_cores * info.num_subcores
    per_worker = total // num_workers
    n_chunks = per_worker // CHUNK
    num_cores = info.num_cores

    mesh = plsc.VectorSubcoreMesh(core_axis_name="c", subcore_axis_name="s")

    @functools.partial(
        pl.kernel,
        mesh=mesh,
        out_type=jax.ShapeDtypeStruct((total // 2, 2 * D_MODEL), jnp.float32),
        scratch_types=[
            pltpu.VMEM((per_worker,), jnp.int32),
            pltpu.VMEM((per_worker,), jnp.int32),
            [pltpu.VMEM((CHUNK, 2 * D_MODEL), jnp.float32) for _ in range(NBUF)],
            [pltpu.SemaphoreType.DMA for _ in range(NBUF)],
            [pltpu.SemaphoreType.DMA for _ in range(NBUF)],
        ],
        compiler_params=pltpu.CompilerParams(needs_layout_passes=False),
    )
    def gather_scale(lut_hbm, x_hbm, out_hbm, kidx_v, poff_v, bufs, gsems, wsems):
        wid = lax.axis_index("s") * num_cores + lax.axis_index("c")
        base = wid * per_worker
        obase = wid * (per_worker // 2)
        pltpu.sync_copy(x_hbm.at[pl.ds(base, per_worker)], kidx_v)

        def prep_body(i, carry):
            for u in range(2):
                s = pl.ds((2 * i + u) * LANES, LANES)
                v = kidx_v[s]
                poff_v[s] = (v & 1) << 6
                kidx_v[s] = v >> 1
            return carry

        lax.fori_loop(0, per_worker // (2 * LANES), prep_body, 0)

        def gather_start(j, b):
            pltpu.async_copy(
                lut_hbm.at[kidx_v.at[pl.ds(j * CHUNK, CHUNK)]], bufs[b], gsems[b]
            )

        def gather_wait(j, b):
            pltpu.make_async_copy(
                lut_hbm.at[kidx_v.at[pl.ds(j * CHUNK, CHUNK)]], bufs[b], gsems[b]
            ).wait()

        def write_start(j, b):
            pltpu.async_copy(
                bufs[b].at[pl.ds(0, CHUNK // 2)],
                out_hbm.at[pl.ds(obase + j * (CHUNK // 2), CHUNK // 2)],
                wsems[b],
            )

        def write_wait(j, b):
            pltpu.make_async_copy(
                bufs[b].at[pl.ds(0, CHUNK // 2)],
                out_hbm.at[pl.ds(obase + j * (CHUNK // 2), CHUNK // 2)],
                wsems[b],
            ).wait()

        iota = jax.lax.iota(jnp.int32, LANES)
        rows_dst_rel = iota >> 1
        cols_dst_rel = (iota & 1) << 6

        def compact_scale(j, b):
            buf = bufs[b]

            def group_body(g, carry):
                j0 = g * LANES
                poff = poff_v[pl.ds(j * CHUNK + j0, LANES)]
                rows_src = iota + j0
                rows_dst = rows_dst_rel + g * (LANES // 2)

                def col_body(cq, carry2):
                    for u in range(4):
                        c = cq * 4 + u
                        v = plsc.load_gather(buf, [rows_src, poff + c])
                        plsc.store_scatter(
                            buf, [rows_dst, cols_dst_rel + c], v * SCALE
                        )
                    return carry2

                lax.fori_loop(0, D_MODEL // 4, col_body, 0)
                return carry

            lax.fori_loop(0, CHUNK // LANES, group_body, 0)

        gather_start(0, 0)
        for j in range(NBUF - 1):
            gather_start(j + 1, j + 1)
            gather_wait(j, j)
            compact_scale(j, j)
            write_start(j, j)

        def steady(ci, carry):
            for b in range(NBUF):
                j = (NBUF - 1) + ci * NBUF + b
                bf = (NBUF - 1 + b) % NBUF
                write_wait(j - (NBUF - 1), b)
                gather_start(j + 1, b)
                gather_wait(j, bf)
                compact_scale(j, bf)
                write_start(j, bf)
            return carry

        lax.fori_loop(0, (n_chunks - NBUF) // NBUF, steady, 0)

        jl = n_chunks - 1
        bl = jl % NBUF
        gather_wait(jl, bl)
        compact_scale(jl, bl)
        write_start(jl, bl)
        for j in range(n_chunks - NBUF, n_chunks):
            write_wait(j, j % NBUF)

    out = gather_scale(lut_pairs, xf)
    return out.reshape(batch_shape + (D_MODEL,))


# speedup vs baseline: 1.8380x; 1.8380x over previous
import functools
import jax
import jax.numpy as jnp
from jax import lax
from jax.experimental import pallas as pl
from jax.experimental.pallas import tpu as pltpu
from jax.experimental.pallas import tpu_sc as plsc

D_MODEL = 64
SCALE = 8.0
LANES = 16
CHUNK = 128
NBUF = 4
UNIT = 8


def kernel(lut, x):
    b_total, seq = x.shape
    n_token = lut.shape[0]
    lutp = jnp.pad(lut, ((0, 0), (0, 2 * D_MODEL - lut.shape[1])))
    xt = x.T.astype(jnp.int32).reshape(seq, b_total // 1024, UNIT, CHUNK)

    info = plsc.get_sparse_core_info()
    num_workers = info.num_cores * info.num_subcores
    num_cores = info.num_cores
    chunks_per_s = b_total // CHUNK
    n_chunks = (b_total * seq) // (CHUNK * num_workers)

    mesh = plsc.VectorSubcoreMesh(core_axis_name="c", subcore_axis_name="s")

    @functools.partial(
        pl.kernel,
        mesh=mesh,
        out_type=jax.ShapeDtypeStruct((seq, D_MODEL, b_total), jnp.float32),
        scratch_types=[
            pltpu.VMEM((UNIT, CHUNK), jnp.int32),
            [pltpu.VMEM((CHUNK,), jnp.int32) for _ in range(NBUF)],
            [pltpu.VMEM((CHUNK, 2 * D_MODEL), jnp.float32) for _ in range(NBUF)],
            [pltpu.VMEM((D_MODEL, CHUNK), jnp.float32) for _ in range(NBUF)],
            [pltpu.SemaphoreType.DMA for _ in range(NBUF)],
            [pltpu.SemaphoreType.DMA for _ in range(NBUF)],
        ],
        compiler_params=pltpu.CompilerParams(needs_layout_passes=False),
    )
    def gather_scale(
        lut_hbm, x_hbm, out_hbm, islot, ichunks, gbufs, tbufs, gsems, wsems
    ):
        wid = lax.axis_index("s") * num_cores + lax.axis_index("c")
        k0 = wid * n_chunks

        def out_coords(j):
            kg = k0 + j
            b0 = pl.multiple_of((kg & (chunks_per_s - 1)) << 7, CHUNK)
            return kg >> 5, b0

        def prep_idx(j, b):
            kg = k0 + j
            cc = kg & (UNIT - 1)

            @pl.when(cc == 0)
            def _():
                pltpu.sync_copy(x_hbm.at[kg >> 5, (kg >> 3) & 3], islot)

            for l in range(CHUNK // LANES):
                sl = pl.ds(l * LANES, LANES)
                ichunks[b][sl] = islot[cc, sl]

        def gather_start(j, b):
            pltpu.async_copy(lut_hbm.at[ichunks[b]], gbufs[b], gsems[b])

        def gather_wait(j, b):
            pltpu.make_async_copy(lut_hbm.at[ichunks[b]], gbufs[b], gsems[b]).wait()

        def write_start(j, b):
            s, b0 = out_coords(j)
            pltpu.async_copy(
                tbufs[b], out_hbm.at[s, :, pl.ds(b0, CHUNK)], wsems[b]
            )

        def write_wait(j, b):
            s, b0 = out_coords(j)
            pltpu.make_async_copy(
                tbufs[b], out_hbm.at[s, :, pl.ds(b0, CHUNK)], wsems[b]
            ).wait()

        iota = jax.lax.iota(jnp.int32, LANES)
        dvecs = [q * LANES + iota for q in range(D_MODEL // LANES)]

        def tscale(b):
            gbuf, tbuf = gbufs[b], tbufs[b]

            def row_body(r2, carry):
                for u in range(2):
                    r = r2 * 2 + u
                    rvec = jnp.full((LANES,), 0, jnp.int32) + r
                    for q in range(D_MODEL // LANES):
                        v = gbuf[r, pl.ds(q * LANES, LANES)]
                        plsc.store_scatter(tbuf, [dvecs[q], rvec], v * SCALE)
                return carry

            lax.fori_loop(0, CHUNK // 2, row_body, 0)

        prep_idx(0, 0)
        gather_start(0, 0)
        prep_idx(1, 1)
        gather_start(1, 1)
        for j in range(2):
            prep_idx(j + 2, (j + 2) % NBUF)
            gather_start(j + 2, (j + 2) % NBUF)
            gather_wait(j, j % NBUF)
            tscale(j % NBUF)
            write_start(j, j % NBUF)

        def steady(ci, carry):
            for u in range(NBUF):
                j = 2 + ci * NBUF + u
                b = (2 + u) % NBUF
                write_wait(j - 2, u % NBUF)
                prep_idx(j + 2, u % NBUF)
                gather_start(j + 2, u % NBUF)
                gather_wait(j, b)
                tscale(b)
                write_start(j, b)
            return carry

        lax.fori_loop(0, (n_chunks - NBUF) // NBUF, steady, 0)

        for j in range(n_chunks - 2, n_chunks):
            gather_wait(j, j % NBUF)
            tscale(j % NBUF)
            write_start(j, j % NBUF)
        for j in range(n_chunks - NBUF, n_chunks):
            write_wait(j, j % NBUF)

    out = gather_scale(lutp, xt)
    return out.transpose(2, 0, 1)


# speedup vs baseline: 2.1401x; 1.1644x over previous
import functools
import jax
import jax.numpy as jnp
from jax import lax
from jax.experimental import pallas as pl
from jax.experimental.pallas import tpu as pltpu
from jax.experimental.pallas import tpu_sc as plsc

D_MODEL = 64
SCALE = 8.0
LANES = 16
CHUNK = 128
NBUF = 4
UNIT = 8
MROW = LANES + 1


def kernel(lut, x):
    b_total, seq = x.shape
    xt = x.T.astype(jnp.int32).reshape(seq, b_total // 1024, UNIT, CHUNK)

    info = plsc.get_sparse_core_info()
    num_workers = info.num_cores * info.num_subcores
    num_cores = info.num_cores
    chunks_per_s = b_total // CHUNK
    n_chunks = (b_total * seq) // (CHUNK * num_workers)

    mesh = plsc.VectorSubcoreMesh(core_axis_name="c", subcore_axis_name="s")

    @functools.partial(
        pl.kernel,
        mesh=mesh,
        out_type=jax.ShapeDtypeStruct(
            (seq, D_MODEL // 8, b_total // CHUNK, 8, CHUNK), jnp.float32
        ),
        scratch_types=[
            pltpu.VMEM((UNIT, CHUNK), jnp.int32),
            [pltpu.VMEM((CHUNK,), jnp.int32) for _ in range(NBUF)],
            [pltpu.VMEM((CHUNK, D_MODEL), jnp.float32) for _ in range(NBUF)],
            [
                pltpu.VMEM((D_MODEL // 8, 8, CHUNK), jnp.float32)
                for _ in range(NBUF)
            ],
            [pltpu.VMEM((LANES * MROW,), jnp.float32) for _ in range(2)],
            [pltpu.SemaphoreType.DMA for _ in range(NBUF)],
            [pltpu.SemaphoreType.DMA for _ in range(NBUF)],
        ],
        compiler_params=pltpu.CompilerParams(
            needs_layout_passes=False, use_tc_tiling_on_sc=False
        ),
    )
    def gather_scale(
        lut_hbm, x_hbm, out_hbm, islot, ichunks, gbufs, tbufs, minis, gsems, wsems
    ):
        wid = lax.axis_index("s") * num_cores + lax.axis_index("c")
        k0 = wid * n_chunks

        def out_coords(j):
            kg = k0 + j
            return kg >> 5, kg & (chunks_per_s - 1)

        def prep_idx(j, b):
            kg = k0 + j
            cc = kg & (UNIT - 1)

            @pl.when(cc == 0)
            def _():
                pltpu.sync_copy(x_hbm.at[kg >> 5, (kg >> 3) & 3], islot)

            for l in range(CHUNK // LANES):
                sl = pl.ds(l * LANES, LANES)
                ichunks[b][sl] = islot[cc, sl]

        def gather_start(j, b):
            pltpu.async_copy(lut_hbm.at[ichunks[b]], gbufs[b], gsems[b])

        def gather_wait(j, b):
            pltpu.make_async_copy(lut_hbm.at[ichunks[b]], gbufs[b], gsems[b]).wait()

        def write_start(j, b):
            s, bg = out_coords(j)
            pltpu.async_copy(tbufs[b], out_hbm.at[s, :, bg], wsems[b])

        def write_wait(j, b):
            s, bg = out_coords(j)
            pltpu.make_async_copy(tbufs[b], out_hbm.at[s, :, bg], wsems[b]).wait()

        iota = jax.lax.iota(jnp.int32, LANES)
        sidx = [iota * MROW + i for i in range(LANES)]

        def tscale(b):
            gbuf, tbuf = gbufs[b], tbufs[b]

            def row_body(r2, carry):
                rb = r2 * LANES
                for d0 in range(D_MODEL // LANES):
                    mini = minis[d0 % 2]
                    for i in range(LANES):
                        v = gbuf[rb + i, pl.ds(d0 * LANES, LANES)]
                        plsc.store_scatter(mini, [sidx[i]], v * SCALE)
                    for d in range(LANES):
                        dd = d0 * LANES + d
                        tbuf[dd >> 3, dd & 7, pl.ds(rb, LANES)] = mini[
                            pl.ds(d * MROW, LANES)
                        ]
                return carry

            lax.fori_loop(0, CHUNK // LANES, row_body, 0)

        prep_idx(0, 0)
        gather_start(0, 0)
        prep_idx(1, 1)
        gather_start(1, 1)
        for j in range(2):
            prep_idx(j + 2, (j + 2) % NBUF)
            gather_start(j + 2, (j + 2) % NBUF)
            gather_wait(j, j % NBUF)
            tscale(j % NBUF)
            write_start(j, j % NBUF)

        def steady(ci, carry):
            for u in range(NBUF):
                j = 2 + ci * NBUF + u
                b = (2 + u) % NBUF
                write_wait(j - 2, u % NBUF)
                prep_idx(j + 2, u % NBUF)
                gather_start(j + 2, u % NBUF)
                gather_wait(j, b)
                tscale(b)
                write_start(j, b)
            return carry

        lax.fori_loop(0, (n_chunks - NBUF) // NBUF, steady, 0)

        for j in range(n_chunks - 2, n_chunks):
            gather_wait(j, j % NBUF)
            tscale(j % NBUF)
            write_start(j, j % NBUF)
        for j in range(n_chunks - NBUF, n_chunks):
            write_wait(j, j % NBUF)

    out = gather_scale(lut, xt)
    return out.transpose(2, 4, 0, 1, 3).reshape(b_total, seq, D_MODEL)
